# baseline (speedup 1.0000x reference)
import jax
import jax.numpy as jnp
from jax.experimental import pallas as pl
from jax.experimental.pallas import tpu as pltpu

_P = 512
_STRIDE = 64
_H, _W = 4000, 6000
_NH = (_H - _P) // _STRIDE + 1
_NW = (_W - _P) // _STRIDE + 1
_NB = 64
_RG = 512


def _rowsum_kernel(x_ref, p1_ref, p2_ref, r1_ref, r2_ref, acc_ref):
    c = pl.program_id(1)
    x = x_ref[0]

    @pl.when(c == 0)
    def _():
        acc_ref[...] = x

    @pl.when(c == 1)
    def _():
        acc_ref[...] = acc_ref[...] + x

    @pl.when(c == 2)
    def _():
        s = acc_ref[...] + x
        s2 = s * s
        for g in range(8):
            blk = s[64 * g:64 * g + 64]
            blk2 = s2[64 * g:64 * g + 64]
            p1_ref[g, :] = jnp.sum(blk, axis=0)
            p2_ref[g, :] = jnp.sum(blk2, axis=0)
            r1_ref[g, :] = blk[0]
            r2_ref[g, :] = blk2[0]


def _select_kernel(p1_ref, p2_ref, r1_ref, r2_ref, m_ref, sc_ref):
    p1 = p1_ref[...]
    p2 = p2_ref[...]
    r1 = r1_ref[...]
    r2 = r2_ref[...]
    q1 = r1[8:8 + _NH] - r1[0:_NH]
    q2 = r2[8:8 + _NH] - r2[0:_NH]
    for b in range(8):
        q1 = q1 + p1[b:b + _NH]
        q2 = q2 + p2[b:b + _NH]
    m = m_ref[...]
    dn = (((1,), (0,)), ((), ()))
    sum1 = jax.lax.dot_general(q1, m, dn,
                               precision=jax.lax.Precision.HIGHEST,
                               preferred_element_type=jnp.float32)
    sum2 = jax.lax.dot_general(q2, m, dn,
                               precision=jax.lax.Precision.HIGHEST,
                               preferred_element_type=jnp.float32)
    n = float(_P * _P)
    std = jnp.sqrt((sum2 - sum1 * sum1 / n) / n)
    minval = jnp.min(std)
    lin = (jax.lax.broadcasted_iota(jnp.int32, (_NH, _NW), 0) * _NW
           + jax.lax.broadcasted_iota(jnp.int32, (_NH, _NW), 1))
    idx = jnp.min(jnp.where(std == minval, lin, jnp.int32(2 ** 30)))
    hblk = idx // _NW
    wb = idx % _NW
    sc_ref[0] = hblk
    sc_ref[1] = wb // 2
    sc_ref[2] = wb % 2


def _crop_kernel(sc_ref, a_ref, b_ref, out_ref):
    par = sc_ref[2]
    av = a_ref[...]
    bv = b_ref[...]
    shifted = jnp.concatenate([av[:, :, 64:], bv[:, :, :64]], axis=2)
    out_ref[...] = jnp.where(par == 1, shifted, av)


def kernel(tensor):
    C, H, W = tensor.shape
    f32 = jnp.float32

    p1, p2, r1, r2 = pl.pallas_call(
        _rowsum_kernel,
        grid=(_H // _RG + 1, 3),
        in_specs=[pl.BlockSpec((1, _RG, W), lambda i, c: (c, i, 0))],
        out_specs=[pl.BlockSpec((8, W), lambda i, c: (i, 0))] * 4,
        out_shape=[jax.ShapeDtypeStruct((_NB, W), f32)] * 4,
        scratch_shapes=[pltpu.VMEM((_RG, W), f32)],
        compiler_params=pltpu.CompilerParams(
            dimension_semantics=("arbitrary", "arbitrary")),
        name="rowblock_sums",
    )(tensor)

    cols = jnp.arange(W, dtype=jnp.int32)[:, None]
    starts = jnp.arange(_NW, dtype=jnp.int32)[None, :] * _STRIDE
    mwin = ((cols >= starts) & (cols < starts + _P)).astype(f32)

    sc = pl.pallas_call(
        _select_kernel,
        in_specs=[
            pl.BlockSpec((_NB, W), lambda: (0, 0)),
            pl.BlockSpec((_NB, W), lambda: (0, 0)),
            pl.BlockSpec((_NB, W), lambda: (0, 0)),
            pl.BlockSpec((_NB, W), lambda: (0, 0)),
            pl.BlockSpec((W, _NW), lambda: (0, 0)),
        ],
        out_specs=pl.BlockSpec(memory_space=pltpu.SMEM),
        out_shape=jax.ShapeDtypeStruct((3,), jnp.int32),
        name="tile_select",
    )(p1, p2, r1, r2, mwin)

    crop = pl.pallas_call(
        _crop_kernel,
        grid_spec=pltpu.PrefetchScalarGridSpec(
            num_scalar_prefetch=1,
            grid=(8, 4),
            in_specs=[
                pl.BlockSpec((3, 64, 128),
                             lambda i, j, s: (0, s[0] + i, s[1] + j)),
                pl.BlockSpec((3, 64, 128),
                             lambda i, j, s: (0, s[0] + i, s[1] + j + 1)),
            ],
            out_specs=pl.BlockSpec((3, 64, 128), lambda i, j, s: (0, i, j)),
        ),
        out_shape=jax.ShapeDtypeStruct((C, _P, _P), f32),
        name="crop",
    )(sc, tensor, tensor)

    return crop


# speedup vs baseline: 1.0713x; 1.0713x over previous
import jax
import jax.numpy as jnp
from jax.experimental import pallas as pl
from jax.experimental.pallas import tpu as pltpu

_P = 512
_STRIDE = 64
_H, _W = 4000, 6000
_NH = (_H - _P) // _STRIDE + 1
_NW = (_W - _P) // _STRIDE + 1
_NB = (_H + 63) // 64


def _rowsum_select_kernel(x_ref, m_ref, sc_ref, p1_ref, p2_ref,
                          r1_ref, r2_ref):
    k = pl.program_id(0)
    x = x_ref[...]
    s = x[0] + x[1] + x[2]
    s2 = s * s
    p1_ref[k, :] = jnp.sum(s, axis=0)
    p2_ref[k, :] = jnp.sum(s2, axis=0)
    r1_ref[k, :] = s[0]
    r2_ref[k, :] = s2[0]

    @pl.when(k == _NB - 1)
    def _select():
        p1 = p1_ref[...]
        p2 = p2_ref[...]
        r1 = r1_ref[...]
        r2 = r2_ref[...]
        q1 = r1[8:8 + _NH] - r1[0:_NH]
        q2 = r2[8:8 + _NH] - r2[0:_NH]
        for b in range(8):
            q1 = q1 + p1[b:b + _NH]
            q2 = q2 + p2[b:b + _NH]
        m = m_ref[...]
        dn = (((1,), (0,)), ((), ()))
        sum1 = jax.lax.dot_general(q1, m, dn,
                                   precision=jax.lax.Precision.HIGHEST,
                                   preferred_element_type=jnp.float32)
        sum2 = jax.lax.dot_general(q2, m, dn,
                                   precision=jax.lax.Precision.HIGHEST,
                                   preferred_element_type=jnp.float32)
        n = float(_P * _P)
        std = jnp.sqrt((sum2 - sum1 * sum1 / n) / n)
        minval = jnp.min(std)
        lin = (jax.lax.broadcasted_iota(jnp.int32, (_NH, _NW), 0) * _NW
               + jax.lax.broadcasted_iota(jnp.int32, (_NH, _NW), 1))
        idx = jnp.min(jnp.where(std == minval, lin, jnp.int32(2 ** 30)))
        hblk = idx // _NW
        wb = idx % _NW
        sc_ref[0] = hblk
        sc_ref[1] = wb // 2
        sc_ref[2] = wb % 2


def _crop_kernel(sc_ref, a_ref, b_ref, out_ref):
    par = sc_ref[2]
    av = a_ref[...]
    bv = b_ref[...]
    shifted = jnp.concatenate([av[:, :, 64:], bv[:, :, :64]], axis=2)
    out_ref[...] = jnp.where(par == 1, shifted, av)


def kernel(tensor):
    C, H, W = tensor.shape
    f32 = jnp.float32

    cols = jnp.arange(W, dtype=jnp.int32)[:, None]
    starts = jnp.arange(_NW, dtype=jnp.int32)[None, :] * _STRIDE
    mwin = ((cols >= starts) & (cols < starts + _P)).astype(f32)

    sc = pl.pallas_call(
        _rowsum_select_kernel,
        grid=(_NB,),
        in_specs=[
            pl.BlockSpec((3, 64, W), lambda k: (0, k, 0)),
            pl.BlockSpec((W, _NW), lambda k: (0, 0)),
        ],
        out_specs=pl.BlockSpec(memory_space=pltpu.SMEM),
        out_shape=jax.ShapeDtypeStruct((3,), jnp.int32),
        scratch_shapes=[pltpu.VMEM((_NB, W), f32)] * 4,
        compiler_params=pltpu.CompilerParams(
            dimension_semantics=("arbitrary",)),
        name="rowsum_select",
    )(tensor, mwin)

    crop = pl.pallas_call(
        _crop_kernel,
        grid_spec=pltpu.PrefetchScalarGridSpec(
            num_scalar_prefetch=1,
            grid=(8, 4),
            in_specs=[
                pl.BlockSpec((3, 64, 128),
                             lambda i, j, s: (0, s[0] + i, s[1] + j)),
                pl.BlockSpec((3, 64, 128),
                             lambda i, j, s: (0, s[0] + i, s[1] + j + 1)),
            ],
            out_specs=pl.BlockSpec((3, 64, 128), lambda i, j, s: (0, i, j)),
        ),
        out_shape=jax.ShapeDtypeStruct((C, _P, _P), f32),
        name="crop",
    )(sc, tensor, tensor)

    return crop


# speedup vs baseline: 1.1930x; 1.1136x over previous
import jax
import jax.numpy as jnp
from jax.experimental import pallas as pl
from jax.experimental.pallas import tpu as pltpu

_P = 512
_STRIDE = 64
_H, _W = 4000, 6000
_NH = (_H - _P) // _STRIDE + 1
_NW = (_W - _P) // _STRIDE + 1
_NB = (_H + 63) // 64


def _rowsum_select_kernel(x_ref, m_ref, sc_ref, p1_ref, p2_ref,
                          r1_ref, r2_ref):
    k = pl.program_id(0)
    x = x_ref[...]
    s = x[0] + x[1] + x[2]
    s2 = s * s
    p1_ref[k, :] = jnp.sum(s, axis=0)
    p2_ref[k, :] = jnp.sum(s2, axis=0)
    r1_ref[k, :] = s[0]
    r2_ref[k, :] = s2[0]

    @pl.when(k == _NB - 1)
    def _select():
        p1 = p1_ref[...]
        p2 = p2_ref[...]
        r1 = r1_ref[...]
        r2 = r2_ref[...]
        q1 = r1[8:8 + _NH] - r1[0:_NH]
        q2 = r2[8:8 + _NH] - r2[0:_NH]
        for b in range(8):
            q1 = q1 + p1[b:b + _NH]
            q2 = q2 + p2[b:b + _NH]
        m = m_ref[...]
        dn = (((1,), (0,)), ((), ()))
        sum1 = jax.lax.dot_general(q1, m, dn,
                                   precision=jax.lax.Precision.HIGHEST,
                                   preferred_element_type=jnp.float32)
        sum2 = jax.lax.dot_general(q2, m, dn,
                                   precision=jax.lax.Precision.HIGHEST,
                                   preferred_element_type=jnp.float32)
        n = float(_P * _P)
        std = jnp.sqrt((sum2 - sum1 * sum1 / n) / n)
        minval = jnp.min(std)
        lin = (jax.lax.broadcasted_iota(jnp.int32, (_NH, _NW), 0) * _NW
               + jax.lax.broadcasted_iota(jnp.int32, (_NH, _NW), 1))
        idx = jnp.min(jnp.where(std == minval, lin, jnp.int32(2 ** 30)))
        hblk = idx // _NW
        wb = idx % _NW
        sc_ref[0] = hblk
        sc_ref[1] = wb // 2
        sc_ref[2] = wb % 2


def _crop_kernel(sc_ref, t0_ref, t1_ref, t2_ref, t3_ref, t4_ref, out_ref):
    par = sc_ref[2]
    c = jnp.concatenate([t0_ref[...], t1_ref[...], t2_ref[...],
                         t3_ref[...], t4_ref[...]], axis=2)
    out_ref[...] = jnp.where(par == 1, c[:, :, 64:576], c[:, :, 0:512])


def kernel(tensor):
    C, H, W = tensor.shape
    f32 = jnp.float32

    cols = jnp.arange(W, dtype=jnp.int32)[:, None]
    starts = jnp.arange(_NW, dtype=jnp.int32)[None, :] * _STRIDE
    mwin = ((cols >= starts) & (cols < starts + _P)).astype(f32)

    sc = pl.pallas_call(
        _rowsum_select_kernel,
        grid=(_NB,),
        in_specs=[
            pl.BlockSpec((3, 64, W), lambda k: (0, k, 0)),
            pl.BlockSpec((W, _NW), lambda k: (0, 0)),
        ],
        out_specs=pl.BlockSpec(memory_space=pltpu.SMEM),
        out_shape=jax.ShapeDtypeStruct((3,), jnp.int32),
        scratch_shapes=[pltpu.VMEM((_NB, W), f32)] * 4,
        compiler_params=pltpu.CompilerParams(
            dimension_semantics=("arbitrary",)),
        name="rowsum_select",
    )(tensor, mwin)

    crop = pl.pallas_call(
        _crop_kernel,
        grid_spec=pltpu.PrefetchScalarGridSpec(
            num_scalar_prefetch=1,
            grid=(8,),
            in_specs=[
                pl.BlockSpec((3, 64, 128),
                             lambda i, s, t=t: (0, s[0] + i, s[1] + t))
                for t in range(5)
            ],
            out_specs=pl.BlockSpec((3, 64, _P), lambda i, s: (0, i, 0)),
        ),
        out_shape=jax.ShapeDtypeStruct((C, _P, _P), f32),
        name="crop",
    )(sc, tensor, tensor, tensor, tensor, tensor)

    return crop
